# speedup vs baseline: 13.0124x; 13.0124x over previous
import jax
import jax.numpy as jnp
from jax.experimental import pallas as pl
from jax.experimental.pallas import tpu as pltpu

_BB = 1024


def _mlp_body(toks_ref, e_ref, w1a_ref, w1b_ref, b1_ref, w2_ref, b2_ref, out_ref):
    toks = toks_ref[...]
    bb = toks.shape[0]
    iota = jax.lax.broadcasted_iota(jnp.int32, (bb, 64), 1)

    ohq = (toks[:, 0:1] == iota).astype(jnp.float32)
    cnt = (toks[:, 1:2] == iota).astype(jnp.float32)
    for t in range(2, 9):
        cnt = cnt + (toks[:, t : t + 1] == iota).astype(jnp.float32)

    e = e_ref[...]
    m1 = jnp.dot(e, w1a_ref[...], preferred_element_type=jnp.float32)
    m2 = jnp.dot(e, w1b_ref[...], preferred_element_type=jnp.float32) * 0.125

    preact = (
        jnp.dot(ohq, m1, preferred_element_type=jnp.float32)
        + jnp.dot(cnt, m2, preferred_element_type=jnp.float32)
        + b1_ref[...]
    )
    h1 = jnp.maximum(preact, 0.0)
    out_ref[...] = (
        jnp.dot(h1, w2_ref[...], preferred_element_type=jnp.float32) + b2_ref[...]
    )


def kernel(seqs, query_tok, embed, W1, b1, W2, b2):
    B = seqs.shape[0]
    toks = jnp.concatenate(
        [query_tok[:, None], seqs[:, 15:23]], axis=1
    ).astype(jnp.int32)
    e64 = embed[:64]
    w1a = W1[:64]
    w1b = W1[64:]

    grid = (B // _BB,)
    return pl.pallas_call(
        _mlp_body,
        grid=grid,
        in_specs=[
            pl.BlockSpec((_BB, 9), lambda i: (i, 0)),
            pl.BlockSpec((64, 64), lambda i: (0, 0)),
            pl.BlockSpec((64, 64), lambda i: (0, 0)),
            pl.BlockSpec((64, 64), lambda i: (0, 0)),
            pl.BlockSpec((1, 64), lambda i: (0, 0)),
            pl.BlockSpec((64, 64), lambda i: (0, 0)),
            pl.BlockSpec((1, 64), lambda i: (0, 0)),
        ],
        out_specs=pl.BlockSpec((_BB, 64), lambda i: (i, 0)),
        out_shape=jax.ShapeDtypeStruct((B, 64), jnp.float32),
    )(toks, e64, w1a, w1b, b1[None, :], W2, b2[None, :])


# speedup vs baseline: 24.8207x; 1.9075x over previous
import jax
import jax.numpy as jnp
from jax import lax
from jax.experimental import pallas as pl
from jax.experimental.pallas import tpu as pltpu

_BB = 2048


def _mlp_body(toks_ref, et_ref, w1at_ref, w1bt_ref, b1_ref, w2_ref, b2_ref, out_ref):
    toks = toks_ref[...]
    bb = toks.shape[1]
    iota = lax.broadcasted_iota(jnp.int32, (64, bb), 0)

    ohq = (toks[0:1, :] == iota).astype(jnp.float32)
    cnt = (toks[1:2, :] == iota).astype(jnp.float32)
    for t in range(2, 9):
        cnt = cnt + (toks[t : t + 1, :] == iota).astype(jnp.float32)

    et = et_ref[...]
    ones_row = jnp.full((1, 64), 1.0, dtype=jnp.float32)
    m1t = (
        jnp.dot(w1at_ref[...], et, preferred_element_type=jnp.float32)
        + jnp.dot(b1_ref[...], ones_row, preferred_element_type=jnp.float32)
    )
    m2t = jnp.dot(w1bt_ref[...], et, preferred_element_type=jnp.float32) * 0.125

    preact_t = jnp.dot(m1t, ohq, preferred_element_type=jnp.float32) + jnp.dot(
        m2t, cnt, preferred_element_type=jnp.float32
    )
    h1t = jnp.maximum(preact_t, 0.0)

    out = lax.dot_general(
        h1t,
        w2_ref[...],
        dimension_numbers=(((0,), (0,)), ((), ())),
        preferred_element_type=jnp.float32,
    )
    out_ref[...] = out + b2_ref[...]


def kernel(seqs, query_tok, embed, W1, b1, W2, b2):
    B = seqs.shape[0]
    toks_t = jnp.concatenate(
        [query_tok[:, None], seqs[:, 15:23]], axis=1
    ).astype(jnp.int32).T
    et = embed[:64].T
    w1at = W1[:64].T
    w1bt = W1[64:].T

    grid = (B // _BB,)
    return pl.pallas_call(
        _mlp_body,
        grid=grid,
        in_specs=[
            pl.BlockSpec((9, _BB), lambda i: (0, i)),
            pl.BlockSpec((64, 64), lambda i: (0, 0)),
            pl.BlockSpec((64, 64), lambda i: (0, 0)),
            pl.BlockSpec((64, 64), lambda i: (0, 0)),
            pl.BlockSpec((64, 1), lambda i: (0, 0)),
            pl.BlockSpec((64, 64), lambda i: (0, 0)),
            pl.BlockSpec((1, 64), lambda i: (0, 0)),
        ],
        out_specs=pl.BlockSpec((_BB, 64), lambda i: (i, 0)),
        out_shape=jax.ShapeDtypeStruct((B, 64), jnp.float32),
    )(toks_t, et, w1at, w1bt, b1[:, None], W2, b2[None, :])


# speedup vs baseline: 26.1988x; 1.0555x over previous
import jax
import jax.numpy as jnp
from jax import lax
from jax.experimental import pallas as pl
from jax.experimental.pallas import tpu as pltpu

_BB = 2048


def _mlp_body(toks_ref, et_ref, w1at_ref, w1bt_ref, b1_ref, w2_ref, b2_ref, out_ref):
    toks = toks_ref[...]
    bb = toks.shape[1]
    iota = lax.broadcasted_iota(jnp.int32, (64, bb), 0).astype(jnp.bfloat16)

    one = jnp.bfloat16(1.0)
    zero = jnp.bfloat16(0.0)
    ohq = jnp.where(toks[0:1, :] == iota, one, zero)
    cnt = jnp.where(toks[1:2, :] == iota, one, zero)
    for t in range(2, 9):
        cnt = cnt + jnp.where(toks[t : t + 1, :] == iota, one, zero)

    et = et_ref[...]
    ones_row = jnp.full((1, 64), 1.0, dtype=jnp.float32)
    m1t = (
        jnp.dot(w1at_ref[...], et, preferred_element_type=jnp.float32)
        + jnp.dot(b1_ref[...], ones_row, preferred_element_type=jnp.float32)
    )
    m2t = jnp.dot(w1bt_ref[...], et, preferred_element_type=jnp.float32) * 0.125

    preact_t = jnp.dot(
        m1t.astype(jnp.bfloat16), ohq, preferred_element_type=jnp.float32
    ) + jnp.dot(
        m2t.astype(jnp.bfloat16), cnt, preferred_element_type=jnp.float32
    )
    h1t = jnp.maximum(preact_t, 0.0)

    out = lax.dot_general(
        h1t,
        w2_ref[...],
        dimension_numbers=(((0,), (0,)), ((), ())),
        preferred_element_type=jnp.float32,
    )
    out_ref[...] = out + b2_ref[...]


def kernel(seqs, query_tok, embed, W1, b1, W2, b2):
    B = seqs.shape[0]
    toks_t = jnp.concatenate(
        [query_tok[:, None], seqs[:, 15:23]], axis=1
    ).astype(jnp.bfloat16).T
    et = embed[:64].T
    w1at = W1[:64].T
    w1bt = W1[64:].T

    grid = (B // _BB,)
    return pl.pallas_call(
        _mlp_body,
        grid=grid,
        in_specs=[
            pl.BlockSpec((9, _BB), lambda i: (0, i)),
            pl.BlockSpec((64, 64), lambda i: (0, 0)),
            pl.BlockSpec((64, 64), lambda i: (0, 0)),
            pl.BlockSpec((64, 64), lambda i: (0, 0)),
            pl.BlockSpec((64, 1), lambda i: (0, 0)),
            pl.BlockSpec((64, 64), lambda i: (0, 0)),
            pl.BlockSpec((1, 64), lambda i: (0, 0)),
        ],
        out_specs=pl.BlockSpec((_BB, 64), lambda i: (i, 0)),
        out_shape=jax.ShapeDtypeStruct((B, 64), jnp.float32),
    )(toks_t, et, w1at, w1bt, b1[:, None], W2, b2[None, :])
